# Initial kernel scaffold; baseline (speedup 1.0000x reference)
import jax
import jax.numpy as jnp
from jax.experimental import pallas as pl


def kernel(x, edge_index, W1, a_src1, a_dst1, b1, W2, a_src2, a_dst2, b2, W_adv, b_adv, W_v1, b_v1, W_v2, b_v2, W_v3, b_v3):
    raise NotImplementedError("write your pallas kernel here")


# speedup vs baseline: 7.1188x; 7.1188x over previous
import functools

import jax
import jax.numpy as jnp
from jax import lax
from jax.experimental import pallas as pl
from jax.experimental.pallas import tpu as pltpu
from jax.experimental.pallas import tpu_sc as plsc

N = 512
NFLAT = 2048
H = 8
E = 16384
F32 = jnp.float32
HIGH = lax.Precision.HIGHEST


def _lin_body(x_ref, w_ref, asrc_ref, adst_ref,
              hw_ref, as_ref, ad_ref, exs_ref, g_ref):
    hw = jnp.dot(x_ref[...], w_ref[...], precision=HIGH,
                 preferred_element_type=F32)
    hw_ref[...] = hw
    front = hw[:N, :]
    als = jnp.dot(front, asrc_ref[...], precision=HIGH,
                  preferred_element_type=F32)
    ald = jnp.dot(front, adst_ref[...], precision=HIGH,
                  preferred_element_type=F32)
    as_ref[...] = als
    ad_ref[...] = ald
    m = jnp.max(als, axis=0, keepdims=True) + jnp.max(ald, axis=0, keepdims=True)
    g = jnp.maximum(m, 0.2 * m)
    g_ref[...] = jnp.concatenate([g, jnp.zeros((1, 8), F32)], axis=1)
    al_self = als + ald
    lr_self = jnp.maximum(al_self, 0.2 * al_self)
    exs_ref[...] = jnp.exp(lr_self - g)


def _lin(xf, W, A_src, A_dst):
    fout = W.shape[1]
    return pl.pallas_call(
        _lin_body,
        out_shape=(
            jax.ShapeDtypeStruct((NFLAT, fout), F32),
            jax.ShapeDtypeStruct((N, H), F32),
            jax.ShapeDtypeStruct((N, H), F32),
            jax.ShapeDtypeStruct((N, H), F32),
            jax.ShapeDtypeStruct((1, 16), F32),
        ),
    )(xf, W, A_src, A_dst)


def _sc_body(src_hbm, dst_hbm, as_hbm, ad_hbm, exs_hbm, g_hbm, c_hbm,
             src_v, dst_v, as_v, ad_v, exs_v, g_v, c_v):
    wid = lax.axis_index("c") * 16 + lax.axis_index("s")
    h = wid // 4
    q = wid % 4
    pltpu.sync_copy(src_hbm, src_v)
    pltpu.sync_copy(dst_hbm, dst_v)
    pltpu.sync_copy(as_hbm, as_v)
    pltpu.sync_copy(ad_hbm, ad_v)
    pltpu.sync_copy(exs_hbm, exs_v)
    pltpu.sync_copy(g_hbm, g_v)

    zero16 = jnp.zeros((16,), F32)

    def zrow(r, carry):
        def zcol(cc, carry2):
            c_v[r, pl.ds(cc * 16, 16)] = zero16
            return carry2
        return lax.fori_loop(0, 32, zcol, carry)
    lax.fori_loop(0, 128, zrow, 0)

    hvec = jnp.full((16,), h, jnp.int32)
    gh = g_v[h, :]
    lo = q * 128

    def edge_step(i, carry):
        s16 = src_v[pl.ds(i * 16, 16)]
        d16 = dst_v[pl.ds(i * 16, 16)]
        a = plsc.load_gather(as_v, [s16 * 8 + hvec])
        b = plsc.load_gather(ad_v, [d16 * 8 + hvec])
        al = a + b
        lr = jnp.maximum(al, 0.2 * al)
        ex = jnp.exp(lr - gh)
        rel = d16 - lo
        msk = (rel >= 0) & (rel < 128)
        relc = jnp.where(msk, rel, 0)
        plsc.addupdate_scatter(c_v, [relc, s16], ex, mask=msk)
        return carry
    lax.fori_loop(0, E // 16, edge_step, 0)

    iota16 = lax.iota(jnp.int32, 16)

    def diag_step(j, carry):
        rel16 = j * 16 + iota16
        d16 = rel16 + lo
        val = plsc.load_gather(exs_v, [d16 * 8 + hvec])
        plsc.addupdate_scatter(c_v, [rel16, d16], val)
        return carry
    lax.fori_loop(0, 8, diag_step, 0)

    pltpu.sync_copy(c_v, c_hbm.at[h, pl.ds(q * 128, 128), :])


@functools.lru_cache(maxsize=None)
def _sc_scatter_kernel():
    return pl.kernel(
        _sc_body,
        out_type=jax.ShapeDtypeStruct((H, N, N), F32),
        mesh=plsc.VectorSubcoreMesh(core_axis_name="c", subcore_axis_name="s"),
        compiler_params=pltpu.CompilerParams(needs_layout_passes=False),
        scratch_types=[
            pltpu.VMEM((E,), jnp.int32),
            pltpu.VMEM((E,), jnp.int32),
            pltpu.VMEM((N * H,), F32),
            pltpu.VMEM((N * H,), F32),
            pltpu.VMEM((N * H,), F32),
            pltpu.VMEM((H, 16), F32),
            pltpu.VMEM((128, N), F32),
        ],
    )


def _sc_scatter(src, dst, als, ald, exs, g_rep):
    return _sc_scatter_kernel()(src, dst, als, ald, exs, g_rep)


def _agg_head(ch, front):
    denom = jnp.sum(ch, axis=1, keepdims=True) + 1e-16
    agg = jnp.dot(ch, front, precision=HIGH, preferred_element_type=F32)
    return agg / denom


def _agg1_body(c_ref, hw_ref, b_ref, out_ref):
    hw = hw_ref[...]
    bias = b_ref[...]
    for h in range(H):
        ch = c_ref[h]
        front = hw[:N, h * 8:(h + 1) * 8]
        out_ref[:N, h * 8:(h + 1) * 8] = jnp.maximum(
            _agg_head(ch, front) + bias[:, h * 8:(h + 1) * 8], 0.0)
    out_ref[N:, :] = jnp.maximum(hw[N:, :] + bias, 0.0)


def _agg1(C, hw, bias):
    return pl.pallas_call(
        _agg1_body,
        out_shape=jax.ShapeDtypeStruct((NFLAT, H * 8), F32),
    )(C, hw, bias)


def _agg2_body(c_ref, hw_ref, b_ref, out_ref):
    hw = hw_ref[...]
    bias = b_ref[...]
    agg = _agg_head(c_ref[0], hw[:N, :])
    out_ref[:N, :] = jnp.maximum(agg + bias, 0.0)
    out_ref[N:, :] = jnp.maximum(hw[N:, :] + bias, 0.0)


def _agg2(C, hw, bias):
    fout = hw.shape[1]
    oc = fout // H
    return pl.pallas_call(
        _agg2_body,
        grid=(H,),
        in_specs=[
            pl.BlockSpec((1, N, N), lambda h: (h, 0, 0)),
            pl.BlockSpec((NFLAT, oc), lambda h: (0, h)),
            pl.BlockSpec((1, oc), lambda h: (0, h)),
        ],
        out_specs=pl.BlockSpec((NFLAT, oc), lambda h: (0, h)),
        out_shape=jax.ShapeDtypeStruct((NFLAT, fout), F32),
    )(C, hw, bias)


_KC = 4096
_KSTEPS = (N * H * 128) // _KC


def _head_body(f_ref, wa_ref, wv_ref, ba_ref, bv1_ref, wv2_ref, bv2_ref,
               wv3_ref, bv3_ref, out_ref, acca, accv):
    k = pl.program_id(0)

    @pl.when(k == 0)
    def _():
        acca[...] = jnp.zeros_like(acca)
        accv[...] = jnp.zeros_like(accv)

    f = f_ref[...]
    acca[...] += jnp.dot(f, wa_ref[...], precision=HIGH,
                         preferred_element_type=F32)
    accv[...] += jnp.dot(f, wv_ref[...], precision=HIGH,
                         preferred_element_type=F32)

    @pl.when(k == _KSTEPS - 1)
    def _():
        adv = jnp.maximum(acca[...] + ba_ref[...], 0.0)
        v = jnp.maximum(accv[...] + bv1_ref[...], 0.0)
        v = jnp.maximum(jnp.dot(v, wv2_ref[...], precision=HIGH,
                                preferred_element_type=F32) + bv2_ref[...], 0.0)
        v = jnp.dot(v, wv3_ref[...], precision=HIGH,
                    preferred_element_type=F32) + bv3_ref[...]
        r = lax.broadcasted_iota(jnp.int32, (18, 18), 0) // 6
        c = lax.broadcasted_iota(jnp.int32, (18, 18), 1) // 6
        mmat = jnp.where(r == c, 1.0 / 6.0, 0.0).astype(F32)
        means = jnp.dot(adv, mmat, precision=HIGH, preferred_element_type=F32)
        out_ref[...] = adv - means + v


def _head(flat, W_adv, b_adv, W_v1, b_v1, W_v2, b_v2, W_v3, b_v3):
    return pl.pallas_call(
        _head_body,
        grid=(_KSTEPS,),
        in_specs=[
            pl.BlockSpec((4, _KC), lambda k: (0, k)),
            pl.BlockSpec((_KC, 18), lambda k: (k, 0)),
            pl.BlockSpec((_KC, 64), lambda k: (k, 0)),
            pl.BlockSpec((1, 18), lambda k: (0, 0)),
            pl.BlockSpec((1, 64), lambda k: (0, 0)),
            pl.BlockSpec((64, 64), lambda k: (0, 0)),
            pl.BlockSpec((1, 64), lambda k: (0, 0)),
            pl.BlockSpec((64, 1), lambda k: (0, 0)),
            pl.BlockSpec((1, 1), lambda k: (0, 0)),
        ],
        out_specs=pl.BlockSpec((4, 18), lambda k: (0, 0)),
        out_shape=jax.ShapeDtypeStruct((4, 18), F32),
        scratch_shapes=[
            pltpu.VMEM((4, 18), F32),
            pltpu.VMEM((4, 64), F32),
        ],
    )(flat, W_adv, W_v1, b_adv, b_v1, W_v2, b_v2, W_v3, b_v3)


def _alpha_mat(a):
    af = a[0]
    o = af.shape[1]
    return (jnp.eye(H, dtype=F32)[:, None, :] * af[:, :, None]).reshape(H * o, H)


def _gat_layer(h_in, src, dst, W, a_src, a_dst, bias, agg_fn):
    hw, als, ald, exs, g16 = _lin(h_in, W, _alpha_mat(a_src), _alpha_mat(a_dst))
    g_rep = jnp.tile(g16[0, :H, None], (1, 16))
    C = _sc_scatter(src, dst, als.reshape(-1), ald.reshape(-1),
                    exs.reshape(-1), g_rep)
    return agg_fn(C, hw, bias.reshape(1, -1))


def kernel(x, edge_index, W1, a_src1, a_dst1, b1, W2, a_src2, a_dst2, b2,
           W_adv, b_adv, W_v1, b_v1, W_v2, b_v2, W_v3, b_v3):
    B = x.shape[0]
    xf = x.reshape(B * N, -1)
    src = edge_index[0]
    dst = edge_index[1]
    h1 = _gat_layer(xf, src, dst, W1, a_src1, a_dst1, b1, _agg1)
    h2 = _gat_layer(h1, src, dst, W2, a_src2, a_dst2, b2, _agg2)
    flat = h2.reshape(B, -1)
    out18 = _head(flat, W_adv, b_adv.reshape(1, -1), W_v1, b_v1.reshape(1, -1),
                  W_v2, b_v2.reshape(1, -1), W_v3, b_v3.reshape(1, 1))
    return out18.reshape(B, 3, 6)


# speedup vs baseline: 13.7564x; 1.9324x over previous
import functools

import jax
import jax.numpy as jnp
from jax import lax
from jax.experimental import pallas as pl
from jax.experimental.pallas import tpu as pltpu
from jax.experimental.pallas import tpu_sc as plsc

N = 512
NFLAT = 2048
H = 8
E = 16384
F32 = jnp.float32
HIGH = lax.Precision.HIGHEST


def _lin_body(x_ref, w_ref, asrc_ref, adst_ref,
              hw_ref, as_ref, ad_ref, exs_ref, g_ref):
    hw = jnp.dot(x_ref[...], w_ref[...], preferred_element_type=F32)
    hw_ref[...] = hw
    front = hw[:N, :]
    als = jnp.dot(front, asrc_ref[...], precision=HIGH,
                  preferred_element_type=F32)
    ald = jnp.dot(front, adst_ref[...], precision=HIGH,
                  preferred_element_type=F32)
    as_ref[...] = als
    ad_ref[...] = ald
    m = jnp.max(als, axis=0, keepdims=True) + jnp.max(ald, axis=0, keepdims=True)
    g = jnp.maximum(m, 0.2 * m)
    g_ref[...] = jnp.concatenate([g, jnp.zeros((1, 8), F32)], axis=1)
    al_self = als + ald
    lr_self = jnp.maximum(al_self, 0.2 * al_self)
    exs_ref[...] = jnp.exp(lr_self - g)


def _lin(xf, W, A_src, A_dst):
    fout = W.shape[1]
    return pl.pallas_call(
        _lin_body,
        out_shape=(
            jax.ShapeDtypeStruct((NFLAT, fout), F32),
            jax.ShapeDtypeStruct((N, H), F32),
            jax.ShapeDtypeStruct((N, H), F32),
            jax.ShapeDtypeStruct((N, H), F32),
            jax.ShapeDtypeStruct((1, 16), F32),
        ),
    )(xf, W, A_src, A_dst)


def _sc_body(src_hbm, dst_hbm, ext_hbm, exs_hbm, c_hbm,
             src_v, dst_v, exh_v, exs_v, c_v):
    wid = lax.axis_index("c") * 16 + lax.axis_index("s")
    h = wid // 4
    q = wid % 4
    pltpu.sync_copy(src_hbm, src_v)
    pltpu.sync_copy(dst_hbm, dst_v)
    pltpu.sync_copy(ext_hbm.at[h], exh_v)
    pltpu.sync_copy(exs_hbm, exs_v)

    zero16 = jnp.zeros((16,), F32)

    def zrow(r, carry):
        def zcol(cc, carry2):
            c_v[r, pl.ds(cc * 16, 16)] = zero16
            return carry2
        return lax.fori_loop(0, 32, zcol, carry)
    lax.fori_loop(0, 128, zrow, 0)

    hvec = jnp.full((16,), h, jnp.int32)
    lo = q * 128

    def edge_step(i, carry):
        s16 = src_v[pl.ds(i * 16, 16)]
        d16 = dst_v[pl.ds(i * 16, 16)]
        ex = exh_v[pl.ds(i * 16, 16)]
        rel = d16 - lo
        msk = (rel >= 0) & (rel < 128)
        relc = jnp.where(msk, rel, 0)
        plsc.addupdate_scatter(c_v, [relc, s16], ex, mask=msk)
        return carry
    lax.fori_loop(0, E // 16, edge_step, 0)

    iota16 = lax.iota(jnp.int32, 16)

    def diag_step(j, carry):
        rel16 = j * 16 + iota16
        d16 = rel16 + lo
        val = plsc.load_gather(exs_v, [d16 * 8 + hvec])
        plsc.addupdate_scatter(c_v, [rel16, d16], val)
        return carry
    lax.fori_loop(0, 8, diag_step, 0)

    pltpu.sync_copy(c_v, c_hbm.at[h, pl.ds(q * 128, 128), :])


@functools.lru_cache(maxsize=None)
def _sc_scatter_kernel():
    return pl.kernel(
        _sc_body,
        out_type=jax.ShapeDtypeStruct((H, N, N), F32),
        mesh=plsc.VectorSubcoreMesh(core_axis_name="c", subcore_axis_name="s"),
        compiler_params=pltpu.CompilerParams(needs_layout_passes=False),
        scratch_types=[
            pltpu.VMEM((E,), jnp.int32),
            pltpu.VMEM((E,), jnp.int32),
            pltpu.VMEM((E,), F32),
            pltpu.VMEM((N * H,), F32),
            pltpu.VMEM((128, N), F32),
        ],
    )


def _sc_scatter(src, dst, ext, exs):
    return _sc_scatter_kernel()(src, dst, ext, exs)


_EC = 2048


def _edge_ex_body(s_ref, d_ref, as_ref, ad_ref, g_ref, ex_ref):
    s = s_ref[...]
    d = d_ref[...]
    ion = lax.broadcasted_iota(jnp.int32, (_EC, N), 1)
    sf = (s == ion).astype(F32)
    df = (d == ion).astype(F32)
    asg = jnp.dot(sf, as_ref[...], precision=HIGH, preferred_element_type=F32)
    adg = jnp.dot(df, ad_ref[...], precision=HIGH, preferred_element_type=F32)
    al = asg + adg
    lr = jnp.maximum(al, 0.2 * al)
    ex_ref[...] = jnp.exp(lr - g_ref[...][:, :H])


def _edge_ex(srcN, dstN, als, ald, g16):
    return pl.pallas_call(
        _edge_ex_body,
        grid=(E // _EC,),
        in_specs=[
            pl.BlockSpec((_EC, 1), lambda e: (e, 0)),
            pl.BlockSpec((_EC, 1), lambda e: (e, 0)),
            pl.BlockSpec((N, H), lambda e: (0, 0)),
            pl.BlockSpec((N, H), lambda e: (0, 0)),
            pl.BlockSpec((1, 16), lambda e: (0, 0)),
        ],
        out_specs=pl.BlockSpec((_EC, H), lambda e: (e, 0)),
        out_shape=jax.ShapeDtypeStruct((E, H), F32),
    )(srcN, dstN, als, ald, g16)


def _agg_head(ch, front):
    denom = jnp.sum(ch, axis=1, keepdims=True) + 1e-16
    agg = jnp.dot(ch, front, precision=HIGH, preferred_element_type=F32)
    return agg / denom


def _agg1_body(c_ref, hw_ref, b_ref, out_ref):
    hw = hw_ref[...]
    bias = b_ref[...]
    for h in range(H):
        ch = c_ref[h]
        front = hw[:N, h * 8:(h + 1) * 8]
        out_ref[:N, h * 8:(h + 1) * 8] = jnp.maximum(
            _agg_head(ch, front) + bias[:, h * 8:(h + 1) * 8], 0.0)
    out_ref[N:, :] = jnp.maximum(hw[N:, :] + bias, 0.0)


def _agg1(C, hw, bias):
    return pl.pallas_call(
        _agg1_body,
        out_shape=jax.ShapeDtypeStruct((NFLAT, H * 8), F32),
    )(C, hw, bias)


def _agg2_body(c_ref, hw_ref, b_ref, out_ref):
    hw = hw_ref[...]
    bias = b_ref[...]
    agg = _agg_head(c_ref[0], hw[:N, :])
    out_ref[:N, :] = jnp.maximum(agg + bias, 0.0)
    out_ref[N:, :] = jnp.maximum(hw[N:, :] + bias, 0.0)


def _agg2(C, hw, bias):
    fout = hw.shape[1]
    oc = fout // H
    return pl.pallas_call(
        _agg2_body,
        grid=(H,),
        in_specs=[
            pl.BlockSpec((1, N, N), lambda h: (h, 0, 0)),
            pl.BlockSpec((NFLAT, oc), lambda h: (0, h)),
            pl.BlockSpec((1, oc), lambda h: (0, h)),
        ],
        out_specs=pl.BlockSpec((NFLAT, oc), lambda h: (0, h)),
        out_shape=jax.ShapeDtypeStruct((NFLAT, fout), F32),
    )(C, hw, bias)


_KC = 16384
_KSTEPS = (N * H * 128) // _KC


def _head_body(f_ref, wa_ref, wv_ref, ba_ref, bv1_ref, wv2_ref, bv2_ref,
               wv3_ref, bv3_ref, out_ref, acca, accv):
    k = pl.program_id(0)

    @pl.when(k == 0)
    def _():
        acca[...] = jnp.zeros_like(acca)
        accv[...] = jnp.zeros_like(accv)

    f = f_ref[...]
    wa = wa_ref[...]
    wv = wv_ref[...]
    for b in range(4):
        fb = f[b:b + 1, :]
        acca[:, b:b + 1] += jnp.sum(wa * fb, axis=1, keepdims=True)
        accv[:, b:b + 1] += jnp.sum(wv * fb, axis=1, keepdims=True)

    @pl.when(k == _KSTEPS - 1)
    def _():
        adv = jnp.maximum(acca[...].T + ba_ref[...], 0.0)
        v = jnp.maximum(accv[...].T + bv1_ref[...], 0.0)
        v = jnp.maximum(jnp.dot(v, wv2_ref[...], precision=HIGH,
                                preferred_element_type=F32) + bv2_ref[...], 0.0)
        v = jnp.dot(v, wv3_ref[...], precision=HIGH,
                    preferred_element_type=F32) + bv3_ref[...]
        r = lax.broadcasted_iota(jnp.int32, (18, 18), 0) // 6
        c = lax.broadcasted_iota(jnp.int32, (18, 18), 1) // 6
        mmat = jnp.where(r == c, 1.0 / 6.0, 0.0).astype(F32)
        means = jnp.dot(adv, mmat, precision=HIGH, preferred_element_type=F32)
        out_ref[...] = adv - means + v


def _head(flat, W_adv, b_adv, W_v1, b_v1, W_v2, b_v2, W_v3, b_v3):
    return pl.pallas_call(
        _head_body,
        grid=(_KSTEPS,),
        in_specs=[
            pl.BlockSpec((4, _KC), lambda k: (0, k)),
            pl.BlockSpec((18, _KC), lambda k: (0, k)),
            pl.BlockSpec((64, _KC), lambda k: (0, k)),
            pl.BlockSpec((1, 18), lambda k: (0, 0)),
            pl.BlockSpec((1, 64), lambda k: (0, 0)),
            pl.BlockSpec((64, 64), lambda k: (0, 0)),
            pl.BlockSpec((1, 64), lambda k: (0, 0)),
            pl.BlockSpec((64, 1), lambda k: (0, 0)),
            pl.BlockSpec((1, 1), lambda k: (0, 0)),
        ],
        out_specs=pl.BlockSpec((4, 18), lambda k: (0, 0)),
        out_shape=jax.ShapeDtypeStruct((4, 18), F32),
        scratch_shapes=[
            pltpu.VMEM((18, 4), F32),
            pltpu.VMEM((64, 4), F32),
        ],
    )(flat, W_adv.T, W_v1.T, b_adv, b_v1, W_v2, b_v2, W_v3, b_v3)


def _alpha_mat(a):
    af = a[0]
    o = af.shape[1]
    return (jnp.eye(H, dtype=F32)[:, None, :] * af[:, :, None]).reshape(H * o, H)


def _gat_layer(h_in, src, dst, W, a_src, a_dst, bias, agg_fn):
    hw, als, ald, exs, g16 = _lin(h_in, W, _alpha_mat(a_src), _alpha_mat(a_dst))
    ex = _edge_ex(src.reshape(E, 1), dst.reshape(E, 1), als, ald, g16)
    C = _sc_scatter(src, dst, ex.T, exs.reshape(-1))
    return agg_fn(C, hw, bias.reshape(1, -1))


def kernel(x, edge_index, W1, a_src1, a_dst1, b1, W2, a_src2, a_dst2, b2,
           W_adv, b_adv, W_v1, b_v1, W_v2, b_v2, W_v3, b_v3):
    B = x.shape[0]
    xf = x.reshape(B * N, -1)
    src = edge_index[0]
    dst = edge_index[1]
    h1 = _gat_layer(xf, src, dst, W1, a_src1, a_dst1, b1, _agg1)
    h2 = _gat_layer(h1, src, dst, W2, a_src2, a_dst2, b2, _agg2)
    flat = h2.reshape(B, -1)
    out18 = _head(flat, W_adv, b_adv.reshape(1, -1), W_v1, b_v1.reshape(1, -1),
                  W_v2, b_v2.reshape(1, -1), W_v3, b_v3.reshape(1, 1))
    return out18.reshape(B, 3, 6)


# speedup vs baseline: 14.9031x; 1.0834x over previous
import functools

import jax
import jax.numpy as jnp
from jax import lax
from jax.experimental import pallas as pl
from jax.experimental.pallas import tpu as pltpu
from jax.experimental.pallas import tpu_sc as plsc

N = 512
NFLAT = 2048
H = 8
E = 16384
F32 = jnp.float32
HIGH = lax.Precision.HIGHEST


def _lin_body(x_ref, w_ref, asrc_ref, adst_ref,
              hw_ref, as_ref, ad_ref, exs_ref, g_ref):
    hw = jnp.dot(x_ref[...], w_ref[...], preferred_element_type=F32)
    hw_ref[...] = hw
    front = hw[:N, :]
    als = jnp.dot(front, asrc_ref[...], precision=HIGH,
                  preferred_element_type=F32)
    ald = jnp.dot(front, adst_ref[...], precision=HIGH,
                  preferred_element_type=F32)
    as_ref[...] = als
    ad_ref[...] = ald
    m = jnp.max(als, axis=0, keepdims=True) + jnp.max(ald, axis=0, keepdims=True)
    g = jnp.maximum(m, 0.2 * m)
    g_ref[...] = jnp.concatenate([g, jnp.zeros((1, 8), F32)], axis=1)
    al_self = als + ald
    lr_self = jnp.maximum(al_self, 0.2 * al_self)
    exs_ref[...] = jnp.exp(lr_self - g)


def _lin(xf, W, A_src, A_dst):
    fout = W.shape[1]
    return pl.pallas_call(
        _lin_body,
        out_shape=(
            jax.ShapeDtypeStruct((NFLAT, fout), F32),
            jax.ShapeDtypeStruct((N, H), F32),
            jax.ShapeDtypeStruct((N, H), F32),
            jax.ShapeDtypeStruct((N, H), F32),
            jax.ShapeDtypeStruct((1, 16), F32),
        ),
    )(xf, W, A_src, A_dst)


def _sc_body(src_hbm, dst_hbm, ext_hbm, exs_hbm, c_hbm,
             src_v, dst_v, exh_v, exs_v, c_v):
    wid = lax.axis_index("c") * 16 + lax.axis_index("s")
    h = wid // 4
    q = wid % 4
    pltpu.sync_copy(src_hbm, src_v)
    pltpu.sync_copy(dst_hbm, dst_v)
    pltpu.sync_copy(ext_hbm.at[h], exh_v)
    pltpu.sync_copy(exs_hbm, exs_v)

    zero16 = jnp.zeros((16,), F32)

    def zrow(r, carry):
        for cc in range(32):
            c_v[r, pl.ds(cc * 16, 16)] = zero16
        return carry
    lax.fori_loop(0, 128, zrow, 0)

    hvec = jnp.full((16,), h, jnp.int32)
    lo = q * 128

    def edge_step(i, carry):
        for u in range(4):
            off = (i * 4 + u) * 16
            s16 = src_v[pl.ds(off, 16)]
            d16 = dst_v[pl.ds(off, 16)]
            ex = exh_v[pl.ds(off, 16)]
            rel = d16 - lo
            msk = (rel >= 0) & (rel < 128)
            relc = jnp.where(msk, rel, 0)
            plsc.addupdate_scatter(c_v, [relc, s16], ex, mask=msk)
        return carry
    lax.fori_loop(0, E // 64, edge_step, 0)

    iota16 = lax.iota(jnp.int32, 16)

    def diag_step(j, carry):
        rel16 = j * 16 + iota16
        d16 = rel16 + lo
        val = plsc.load_gather(exs_v, [d16 * 8 + hvec])
        plsc.addupdate_scatter(c_v, [rel16, d16], val)
        return carry
    lax.fori_loop(0, 8, diag_step, 0)

    pltpu.sync_copy(c_v, c_hbm.at[h, pl.ds(q * 128, 128), :])


@functools.lru_cache(maxsize=None)
def _sc_scatter_kernel():
    return pl.kernel(
        _sc_body,
        out_type=jax.ShapeDtypeStruct((H, N, N), F32),
        mesh=plsc.VectorSubcoreMesh(core_axis_name="c", subcore_axis_name="s"),
        compiler_params=pltpu.CompilerParams(needs_layout_passes=False),
        scratch_types=[
            pltpu.VMEM((E,), jnp.int32),
            pltpu.VMEM((E,), jnp.int32),
            pltpu.VMEM((E,), F32),
            pltpu.VMEM((N * H,), F32),
            pltpu.VMEM((128, N), F32),
        ],
    )


def _sc_scatter(src, dst, ext, exs):
    return _sc_scatter_kernel()(src, dst, ext, exs)


_EC = 2048


def _edge_ex_body(s_ref, d_ref, as_ref, ad_ref, g_ref, ex_ref):
    s = s_ref[...]
    d = d_ref[...]
    ion = lax.broadcasted_iota(jnp.int32, (_EC, N), 1)
    sf = (s == ion).astype(F32)
    df = (d == ion).astype(F32)
    asg = jnp.dot(sf, as_ref[...], precision=HIGH, preferred_element_type=F32)
    adg = jnp.dot(df, ad_ref[...], precision=HIGH, preferred_element_type=F32)
    al = asg + adg
    lr = jnp.maximum(al, 0.2 * al)
    ex_ref[...] = jnp.exp(lr - g_ref[...][:, :H])


def _edge_ex(srcN, dstN, als, ald, g16):
    return pl.pallas_call(
        _edge_ex_body,
        grid=(E // _EC,),
        in_specs=[
            pl.BlockSpec((_EC, 1), lambda e: (e, 0)),
            pl.BlockSpec((_EC, 1), lambda e: (e, 0)),
            pl.BlockSpec((N, H), lambda e: (0, 0)),
            pl.BlockSpec((N, H), lambda e: (0, 0)),
            pl.BlockSpec((1, 16), lambda e: (0, 0)),
        ],
        out_specs=pl.BlockSpec((_EC, H), lambda e: (e, 0)),
        out_shape=jax.ShapeDtypeStruct((E, H), F32),
    )(srcN, dstN, als, ald, g16)


def _agg_head(ch, front):
    denom = jnp.sum(ch, axis=1, keepdims=True) + 1e-16
    agg = jnp.dot(ch, front, precision=HIGH, preferred_element_type=F32)
    return agg / denom


def _agg1_body(c_ref, hw_ref, b_ref, out_ref):
    hw = hw_ref[...]
    bias = b_ref[...]
    for h in range(H):
        ch = c_ref[h]
        front = hw[:N, h * 8:(h + 1) * 8]
        out_ref[:N, h * 8:(h + 1) * 8] = jnp.maximum(
            _agg_head(ch, front) + bias[:, h * 8:(h + 1) * 8], 0.0)
    out_ref[N:, :] = jnp.maximum(hw[N:, :] + bias, 0.0)


def _agg1(C, hw, bias):
    return pl.pallas_call(
        _agg1_body,
        out_shape=jax.ShapeDtypeStruct((NFLAT, H * 8), F32),
    )(C, hw, bias)


def _agg2_body(c_ref, hw_ref, b_ref, out_ref):
    hw = hw_ref[...]
    bias = b_ref[...]
    agg = _agg_head(c_ref[0], hw[:N, :])
    out_ref[:N, :] = jnp.maximum(agg + bias, 0.0)
    out_ref[N:, :] = jnp.maximum(hw[N:, :] + bias, 0.0)


def _agg2(C, hw, bias):
    fout = hw.shape[1]
    oc = fout // H
    return pl.pallas_call(
        _agg2_body,
        grid=(H,),
        in_specs=[
            pl.BlockSpec((1, N, N), lambda h: (h, 0, 0)),
            pl.BlockSpec((NFLAT, oc), lambda h: (0, h)),
            pl.BlockSpec((1, oc), lambda h: (0, h)),
        ],
        out_specs=pl.BlockSpec((NFLAT, oc), lambda h: (0, h)),
        out_shape=jax.ShapeDtypeStruct((NFLAT, fout), F32),
    )(C, hw, bias)


_KC = 16384
_KSTEPS = (N * H * 128) // _KC


def _head_body(f_ref, wa_ref, wv_ref, ba_ref, bv1_ref, wv2_ref, bv2_ref,
               wv3_ref, bv3_ref, out_ref, acca, accv):
    k = pl.program_id(0)

    @pl.when(k == 0)
    def _():
        acca[...] = jnp.zeros_like(acca)
        accv[...] = jnp.zeros_like(accv)

    f = f_ref[...]
    wa = wa_ref[...]
    wv = wv_ref[...]
    for b in range(4):
        fb = f[b:b + 1, :]
        acca[:, b:b + 1] += jnp.sum(wa * fb, axis=1, keepdims=True)
        accv[:, b:b + 1] += jnp.sum(wv * fb, axis=1, keepdims=True)

    @pl.when(k == _KSTEPS - 1)
    def _():
        adv = jnp.maximum(acca[...].T + ba_ref[...], 0.0)
        v = jnp.maximum(accv[...].T + bv1_ref[...], 0.0)
        v = jnp.maximum(jnp.dot(v, wv2_ref[...], precision=HIGH,
                                preferred_element_type=F32) + bv2_ref[...], 0.0)
        v = jnp.dot(v, wv3_ref[...], precision=HIGH,
                    preferred_element_type=F32) + bv3_ref[...]
        r = lax.broadcasted_iota(jnp.int32, (18, 18), 0) // 6
        c = lax.broadcasted_iota(jnp.int32, (18, 18), 1) // 6
        mmat = jnp.where(r == c, 1.0 / 6.0, 0.0).astype(F32)
        means = jnp.dot(adv, mmat, precision=HIGH, preferred_element_type=F32)
        out_ref[...] = adv - means + v


def _head(flat, W_adv, b_adv, W_v1, b_v1, W_v2, b_v2, W_v3, b_v3):
    return pl.pallas_call(
        _head_body,
        grid=(_KSTEPS,),
        in_specs=[
            pl.BlockSpec((4, _KC), lambda k: (0, k)),
            pl.BlockSpec((18, _KC), lambda k: (0, k)),
            pl.BlockSpec((64, _KC), lambda k: (0, k)),
            pl.BlockSpec((1, 18), lambda k: (0, 0)),
            pl.BlockSpec((1, 64), lambda k: (0, 0)),
            pl.BlockSpec((64, 64), lambda k: (0, 0)),
            pl.BlockSpec((1, 64), lambda k: (0, 0)),
            pl.BlockSpec((64, 1), lambda k: (0, 0)),
            pl.BlockSpec((1, 1), lambda k: (0, 0)),
        ],
        out_specs=pl.BlockSpec((4, 18), lambda k: (0, 0)),
        out_shape=jax.ShapeDtypeStruct((4, 18), F32),
        scratch_shapes=[
            pltpu.VMEM((18, 4), F32),
            pltpu.VMEM((64, 4), F32),
        ],
    )(flat, W_adv.T, W_v1.T, b_adv, b_v1, W_v2, b_v2, W_v3, b_v3)


def _alpha_mat(a):
    af = a[0]
    o = af.shape[1]
    return (jnp.eye(H, dtype=F32)[:, None, :] * af[:, :, None]).reshape(H * o, H)


def _gat_layer(h_in, src, dst, W, a_src, a_dst, bias, agg_fn):
    hw, als, ald, exs, g16 = _lin(h_in, W, _alpha_mat(a_src), _alpha_mat(a_dst))
    ex = _edge_ex(src.reshape(E, 1), dst.reshape(E, 1), als, ald, g16)
    C = _sc_scatter(src, dst, ex.T, exs.reshape(-1))
    return agg_fn(C, hw, bias.reshape(1, -1))


def kernel(x, edge_index, W1, a_src1, a_dst1, b1, W2, a_src2, a_dst2, b2,
           W_adv, b_adv, W_v1, b_v1, W_v2, b_v2, W_v3, b_v3):
    B = x.shape[0]
    xf = x.reshape(B * N, -1)
    src = edge_index[0]
    dst = edge_index[1]
    h1 = _gat_layer(xf, src, dst, W1, a_src1, a_dst1, b1, _agg1)
    h2 = _gat_layer(h1, src, dst, W2, a_src2, a_dst2, b2, _agg2)
    flat = h2.reshape(B, -1)
    out18 = _head(flat, W_adv, b_adv.reshape(1, -1), W_v1, b_v1.reshape(1, -1),
                  W_v2, b_v2.reshape(1, -1), W_v3, b_v3.reshape(1, 1))
    return out18.reshape(B, 3, 6)
